# Initial kernel scaffold; baseline (speedup 1.0000x reference)
import jax
import jax.numpy as jnp
from jax.experimental import pallas as pl


def kernel(x, edge_index, W1, b1, W2, b2):
    raise NotImplementedError("write your pallas kernel here")


# speedup vs baseline: 12.4980x; 12.4980x over previous
import functools

import jax
import jax.numpy as jnp
from jax import lax
from jax.experimental import pallas as pl
from jax.experimental.pallas import tpu as pltpu
from jax.experimental.pallas import tpu_sc as plsc

N = 10000
E = 320000
D = 128
NC = 2
NS = 16
NW = NC * NS
NPAD = 10240
RPS = NPAD // NS
CHUNK = 128
NCHUNK = -(-E // (NW * CHUNK))
EPAD = NW * NCHUNK * CHUNK
DEGW = 128

_HIGH = lax.Precision.HIGHEST


def _sc_mesh():
    return plsc.VectorSubcoreMesh(core_axis_name="c", subcore_axis_name="s")


def _sc_degree(dst_idx):
    @functools.partial(
        pl.kernel,
        out_type=jax.ShapeDtypeStruct((NC, NPAD, DEGW), jnp.float32),
        mesh=_sc_mesh(),
        scratch_types=[
            pltpu.VMEM((NCHUNK, CHUNK), jnp.int32),
            pltpu.VMEM((CHUNK, DEGW), jnp.float32),
            pltpu.VMEM((CHUNK, DEGW), jnp.float32),
            pltpu.VMEM_SHARED((NPAD, DEGW), jnp.float32),
        ],
    )
    def k(d_hbm, out_hbm, didx, zbuf, ones, acc):
        c = lax.axis_index("c")
        s = lax.axis_index("s")
        w = c * NS + s
        pltpu.sync_copy(d_hbm.at[w], didx)

        @pl.loop(0, CHUNK)
        def _(i):
            zbuf.at[i, pl.ds(0, DEGW)][...] = jnp.zeros((DEGW,), jnp.float32)
            ones.at[i, pl.ds(0, DEGW)][...] = jnp.full((DEGW,), 1.0, jnp.float32)

        @pl.loop(0, RPS, step=CHUNK)
        def _(r):
            pltpu.sync_copy(zbuf, acc.at[pl.ds(s * RPS + r, CHUNK)])

        plsc.subcore_barrier()

        @pl.loop(0, NCHUNK)
        def _(j):
            pltpu.sync_copy(ones, acc.at[didx.at[j]], add=True)

        plsc.subcore_barrier()
        pltpu.sync_copy(acc.at[pl.ds(s * RPS, RPS)],
                        out_hbm.at[c, pl.ds(s * RPS, RPS)])

    return k(dst_idx)


def _sc_propagate(u_pad, src_idx, dst_idx):
    @functools.partial(
        pl.kernel,
        out_type=jax.ShapeDtypeStruct((NC, NPAD, D), jnp.float32),
        mesh=_sc_mesh(),
        scratch_types=[
            pltpu.VMEM((NCHUNK, CHUNK), jnp.int32),
            pltpu.VMEM((NCHUNK, CHUNK), jnp.int32),
            pltpu.VMEM((CHUNK, D), jnp.float32),
            pltpu.VMEM_SHARED((NPAD, D), jnp.float32),
        ],
    )
    def k(u_hbm, s_hbm, d_hbm, out_hbm, sidx, didx, rows, acc):
        c = lax.axis_index("c")
        s = lax.axis_index("s")
        w = c * NS + s
        pltpu.sync_copy(s_hbm.at[w], sidx)
        pltpu.sync_copy(d_hbm.at[w], didx)

        @pl.loop(0, CHUNK)
        def _(i):
            @pl.loop(0, D, step=16)
            def _(j):
                rows.at[i, pl.ds(j, 16)][...] = jnp.zeros((16,), jnp.float32)

        @pl.loop(0, RPS, step=CHUNK)
        def _(r):
            pltpu.sync_copy(rows, acc.at[pl.ds(s * RPS + r, CHUNK)])

        plsc.subcore_barrier()

        @pl.loop(0, NCHUNK)
        def _(j):
            pltpu.sync_copy(u_hbm.at[sidx.at[j]], rows)
            pltpu.sync_copy(rows, acc.at[didx.at[j]], add=True)

        plsc.subcore_barrier()
        pltpu.sync_copy(acc.at[pl.ds(s * RPS, RPS)],
                        out_hbm.at[c, pl.ds(s * RPS, RPS)])

    return k(u_pad, src_idx, dst_idx)


_BLK = 512
_GRID = (NPAD // _BLK,)


def _tc_matmul(x, w):
    def body(x_ref, w_ref, o_ref):
        o_ref[...] = jnp.dot(x_ref[...], w_ref[...], precision=_HIGH,
                             preferred_element_type=jnp.float32)

    return pl.pallas_call(
        body,
        grid=_GRID,
        in_specs=[pl.BlockSpec((_BLK, D), lambda i: (i, 0)),
                  pl.BlockSpec((D, D), lambda i: (0, 0))],
        out_specs=pl.BlockSpec((_BLK, D), lambda i: (i, 0)),
        out_shape=jax.ShapeDtypeStruct((NPAD, D), jnp.float32),
    )(x, w)


def _tc_scale(xw, degp):
    def body(xw_ref, deg_ref, u_ref, dinv_ref):
        deg = deg_ref[0][:, 0:1] + deg_ref[1][:, 0:1] + 1.0
        dinv = lax.rsqrt(deg)
        u_ref[...] = xw_ref[...] * dinv
        dinv_ref[...] = jnp.broadcast_to(dinv, (_BLK, DEGW))

    return pl.pallas_call(
        body,
        grid=_GRID,
        in_specs=[pl.BlockSpec((_BLK, D), lambda i: (i, 0)),
                  pl.BlockSpec((NC, _BLK, DEGW), lambda i: (0, i, 0))],
        out_specs=[pl.BlockSpec((_BLK, D), lambda i: (i, 0)),
                   pl.BlockSpec((_BLK, DEGW), lambda i: (i, 0))],
        out_shape=[jax.ShapeDtypeStruct((NPAD, D), jnp.float32),
                   jax.ShapeDtypeStruct((NPAD, DEGW), jnp.float32)],
    )(xw, degp)


def _tc_mid(sp, u1, dinv16, w2, b1):
    def body(sp_ref, u_ref, dinv_ref, w_ref, b_ref, o_ref):
        dinv = dinv_ref[:, 0:1]
        pre = dinv * (sp_ref[0] + sp_ref[1] + u_ref[...]) + b_ref[...]
        h = jnp.maximum(pre, 0.0)
        o_ref[...] = jnp.dot(h, w_ref[...], precision=_HIGH,
                             preferred_element_type=jnp.float32) * dinv

    return pl.pallas_call(
        body,
        grid=_GRID,
        in_specs=[pl.BlockSpec((NC, _BLK, D), lambda i: (0, i, 0)),
                  pl.BlockSpec((_BLK, D), lambda i: (i, 0)),
                  pl.BlockSpec((_BLK, DEGW), lambda i: (i, 0)),
                  pl.BlockSpec((D, D), lambda i: (0, 0)),
                  pl.BlockSpec((1, D), lambda i: (0, 0))],
        out_specs=pl.BlockSpec((_BLK, D), lambda i: (i, 0)),
        out_shape=jax.ShapeDtypeStruct((NPAD, D), jnp.float32),
    )(sp, u1, dinv16, w2, b1)


def _tc_final(sp, u2, dinv16, b2):
    def body(sp_ref, u_ref, dinv_ref, b_ref, o_ref):
        dinv = dinv_ref[:, 0:1]
        o_ref[...] = dinv * (sp_ref[0] + sp_ref[1] + u_ref[...]) + b_ref[...]

    return pl.pallas_call(
        body,
        grid=_GRID,
        in_specs=[pl.BlockSpec((NC, _BLK, D), lambda i: (0, i, 0)),
                  pl.BlockSpec((_BLK, D), lambda i: (i, 0)),
                  pl.BlockSpec((_BLK, DEGW), lambda i: (i, 0)),
                  pl.BlockSpec((1, D), lambda i: (0, 0))],
        out_specs=pl.BlockSpec((_BLK, D), lambda i: (i, 0)),
        out_shape=jax.ShapeDtypeStruct((NPAD, D), jnp.float32),
    )(sp, u2, dinv16, b2)


def kernel(x, edge_index, W1, b1, W2, b2):
    src = edge_index[0]
    dst = edge_index[1]
    pad = jnp.full((EPAD - E,), N, jnp.int32)
    src_r = jnp.concatenate([src, pad]).reshape(NW, NCHUNK, CHUNK)
    dst_r = jnp.concatenate([dst, pad]).reshape(NW, NCHUNK, CHUNK)
    x_pad = jnp.concatenate([x, jnp.zeros((NPAD - N, D), x.dtype)], axis=0)
    b1r = b1.reshape(1, D)
    b2r = b2.reshape(1, D)

    degp = _sc_degree(dst_r)
    xw1 = _tc_matmul(x_pad, W1)
    u1, dinv16 = _tc_scale(xw1, degp)

    s1 = _sc_propagate(u1, src_r, dst_r)
    u2 = _tc_mid(s1, u1, dinv16, W2, b1r)

    s2 = _sc_propagate(u2, src_r, dst_r)
    z = _tc_final(s2, u2, dinv16, b2r)

    return z[:N]
